# Initial kernel scaffold; baseline (speedup 1.0000x reference)
import jax
import jax.numpy as jnp
from jax.experimental import pallas as pl


def kernel(nodes, edge_index, local_features, W, b, a):
    raise NotImplementedError("write your pallas kernel here")


# speedup vs baseline: 8.4111x; 8.4111x over previous
import functools

import jax
import jax.numpy as jnp
from jax import lax
from jax.experimental import pallas as pl
from jax.experimental.pallas import tpu as pltpu
from jax.experimental.pallas import tpu_sc as plsc

_N = 10000
_E = 320000
_D = 128
_SLOPE = 0.1

_NC = 2
_NS = 16
_NW = _NC * _NS
_EPW = _E // _NW
_C = 80
_NCHUNK = _EPW // _C
_GRP = _C // 16
_ZROWS = 640


def _embed_body(x_ref, w_ref, b_ref, a1_ref, a2_ref, emb_ref, s1_ref, s2_ref):
    emb = jnp.dot(x_ref[...], w_ref[...], preferred_element_type=jnp.float32)
    emb = emb + b_ref[...]
    emb_ref[...] = emb
    s1_ref[...] = jnp.dot(emb, a1_ref[...], preferred_element_type=jnp.float32)
    s2_ref[...] = jnp.dot(emb, a2_ref[...], preferred_element_type=jnp.float32)


def _combine_body(pe_ref, ph_ref, emb_ref, s1_ref, s2_ref, out_ref):
    x = s1_ref[...] + s2_ref[...]
    hs = jnp.exp(jnp.maximum(x, _SLOPE * x))
    num = pe_ref[0, :, :] + pe_ref[1, :, :] + hs * emb_ref[...]
    den = ph_ref[0, :, :] + ph_ref[1, :, :] + hs
    out_ref[...] = num / den


def _sc_agg_body(emb_hbm, s1_hbm, s2_hbm, src_hbm, dst_hbm,
                 out_emb, out_h,
                 s1_v, s2_v, src_v, dst_v, rows_v, h_v, acc_emb, acc_h):
    c = lax.axis_index("c")
    s = lax.axis_index("s")
    wid = s * _NC + c

    if True:
        zero16 = jnp.zeros((16,), jnp.float32)

        def zrow(i, _):
            for j in range(_D // 16):
                rows_v[i, pl.ds(j * 16, 16)] = zero16
            return 0

        lax.fori_loop(0, _C, zrow, 0)
        for g in range(_GRP):
            h_v[pl.ds(g * 16, 16)] = zero16

        ncopies = jnp.minimum(_ZROWS, jnp.maximum(0, _N - s * _ZROWS)) // _C

        def zacc(i, _):
            off = s * _ZROWS + i * _C
            pltpu.sync_copy(rows_v, acc_emb.at[pl.ds(off, _C), :])
            pltpu.sync_copy(h_v, acc_h.at[pl.ds(off, _C)])
            return 0

        lax.fori_loop(0, ncopies, zacc, 0)

        pltpu.sync_copy(s1_hbm, s1_v)
        pltpu.sync_copy(s2_hbm, s2_v)
        plsc.subcore_barrier()

        def chunk(k, _):
            base = wid * _EPW + k * _C
            pltpu.sync_copy(src_hbm.at[pl.ds(base, _C)], src_v)
            pltpu.sync_copy(dst_hbm.at[pl.ds(base, _C)], dst_v)
            pltpu.sync_copy(emb_hbm.at[dst_v], rows_v)
            for g in range(_GRP):
                srcv = src_v[pl.ds(g * 16, 16)]
                dstv = dst_v[pl.ds(g * 16, 16)]
                s1g = plsc.load_gather(s1_v, [srcv])
                s2g = plsc.load_gather(s2_v, [dstv])
                x = s1g + s2g
                h = jnp.exp(jnp.maximum(x, _SLOPE * x))
                h_v[pl.ds(g * 16, 16)] = h
                for e in range(16):
                    he = h.at[jnp.full((16,), e, jnp.int32)].get(
                        mode="promise_in_bounds")
                    for j in range(_D // 16):
                        sl = pl.ds(j * 16, 16)
                        rows_v[g * 16 + e, sl] = rows_v[g * 16 + e, sl] * he
            pltpu.sync_copy(rows_v, acc_emb.at[src_v], add=True)
            pltpu.sync_copy(h_v, acc_h.at[src_v], add=True)
            return 0

        lax.fori_loop(0, _NCHUNK, chunk, 0)
        plsc.subcore_barrier()

        def copy_out(i, _):
            off = s * _ZROWS + i * _C
            pltpu.sync_copy(acc_emb.at[pl.ds(off, _C), :], out_emb.at[c, pl.ds(off, _C), :])
            pltpu.sync_copy(acc_h.at[pl.ds(off, _C)], h_v)
            pltpu.sync_copy(h_v, out_h.at[pl.ds(c * _N + off, _C)])
            return 0

        lax.fori_loop(0, ncopies, copy_out, 0)


_sc_agg = functools.partial(
    pl.kernel,
    out_type=[
        jax.ShapeDtypeStruct((_NC, _N, _D), jnp.float32),
        jax.ShapeDtypeStruct((_NC * _N,), jnp.float32),
    ],
    mesh=plsc.VectorSubcoreMesh(core_axis_name="c", subcore_axis_name="s"),
    compiler_params=pltpu.CompilerParams(needs_layout_passes=False),
    scratch_types=[
        pltpu.VMEM((_N,), jnp.float32),
        pltpu.VMEM((_N,), jnp.float32),
        pltpu.VMEM((_C,), jnp.int32),
        pltpu.VMEM((_C,), jnp.int32),
        pltpu.VMEM((_C, _D), jnp.float32),
        pltpu.VMEM((_C,), jnp.float32),
        pltpu.VMEM_SHARED((_N, _D), jnp.float32),
        pltpu.VMEM_SHARED((_N,), jnp.float32),
    ],
)(_sc_agg_body)


def kernel(nodes, edge_index, local_features, W, b, a):
    x = local_features.astype(jnp.float32)
    W = W.astype(jnp.float32)
    b2 = b.astype(jnp.float32).reshape(1, _D)
    a1 = a.astype(jnp.float32)[:_D].reshape(_D, 1)
    a2 = a.astype(jnp.float32)[_D:].reshape(_D, 1)
    src = edge_index[0].astype(jnp.int32)
    dst = edge_index[1].astype(jnp.int32)

    emb, s1, s2 = pl.pallas_call(
        _embed_body,
        out_shape=[
            jax.ShapeDtypeStruct((_N, _D), jnp.float32),
            jax.ShapeDtypeStruct((_N, 1), jnp.float32),
            jax.ShapeDtypeStruct((_N, 1), jnp.float32),
        ],
    )(x, W, b2, a1, a2)

    s1f = s1.reshape(_N)
    s2f = s2.reshape(_N)
    part_emb, part_h = _sc_agg(emb, s1f, s2f, src, dst)

    out = pl.pallas_call(
        _combine_body,
        out_shape=jax.ShapeDtypeStruct((_N, _D), jnp.float32),
    )(part_emb, part_h.reshape(_NC, _N, 1), emb, s1, s2)
    return out


# speedup vs baseline: 15.9443x; 1.8956x over previous
import functools

import jax
import jax.numpy as jnp
from jax import lax
from jax.experimental import pallas as pl
from jax.experimental.pallas import tpu as pltpu
from jax.experimental.pallas import tpu_sc as plsc

_N = 10000
_E = 320000
_D = 128
_SLOPE = 0.1

_NC = 2
_NS = 16
_NW = _NC * _NS
_EPW = _E // _NW
_C = 80
_NCHUNK = _EPW // _C
_GRP = _C // 16
_ZROWS = 640


def _embed_body(x_ref, w_ref, b_ref, a1_ref, a2_ref, emb_ref, s1_ref, s2_ref):
    emb = jnp.dot(x_ref[...], w_ref[...], preferred_element_type=jnp.float32)
    emb = emb + b_ref[...]
    emb_ref[...] = emb
    s1_ref[...] = jnp.dot(emb, a1_ref[...], preferred_element_type=jnp.float32)
    s2_ref[...] = jnp.dot(emb, a2_ref[...], preferred_element_type=jnp.float32)


def _combine_body(pe_ref, ph_ref, emb_ref, s1_ref, s2_ref, out_ref):
    x = s1_ref[...] + s2_ref[...]
    hs = jnp.exp(jnp.maximum(x, _SLOPE * x))
    num = pe_ref[0, :, :] + pe_ref[1, :, :] + hs * emb_ref[...]
    den = ph_ref[0, :, :] + ph_ref[1, :, :] + hs
    out_ref[...] = num / den


def _sc_agg_body(emb_hbm, s1_hbm, s2_hbm, sd_hbm,
                 out_emb, out_h,
                 sd_a, src_c, dst_c, s1c, s2c, rows0, rows1, h_v,
                 acc_emb, acc_h, sem0, sem1):
    c = lax.axis_index("c")
    s = lax.axis_index("s")
    wid = s * _NC + c
    rows = (rows0, rows1)
    sems = (sem0, sem1)

    zero16 = jnp.zeros((16,), jnp.float32)

    def zrow(i, _):
        for j in range(_D // 16):
            rows0[i, pl.ds(j * 16, 16)] = zero16
        return 0

    lax.fori_loop(0, _C, zrow, 0)
    for g in range(_GRP):
        h_v[pl.ds(g * 16, 16)] = zero16

    ncopies = jnp.minimum(_ZROWS, jnp.maximum(0, _N - s * _ZROWS)) // _C

    def zacc(i, _):
        off = s * _ZROWS + i * _C
        pltpu.sync_copy(rows0, acc_emb.at[pl.ds(off, _C), :])
        pltpu.sync_copy(h_v, acc_h.at[pl.ds(off, _C)])
        return 0

    lax.fori_loop(0, ncopies, zacc, 0)

    pltpu.sync_copy(sd_hbm.at[wid], sd_a)
    plsc.subcore_barrier()

    def unpack(k, b):
        for g in range(_GRP):
            sl = pl.ds(g * 16, 16)
            w = sd_a[k, sl]
            src_c[b, sl] = lax.shift_right_logical(w, 16)
            dst_c[b, sl] = jnp.bitwise_and(w, 0xFFFF)

    def start_gathers(b):
        pltpu.async_copy(emb_hbm.at[dst_c.at[b]], rows[b], sems[b])
        pltpu.async_copy(s1_hbm.at[src_c.at[b]], s1c.at[b], sems[b])
        pltpu.async_copy(s2_hbm.at[dst_c.at[b]], s2c.at[b], sems[b])

    def wait_gathers(b):
        pltpu.make_async_copy(emb_hbm.at[dst_c.at[b]], rows[b], sems[b]).wait()
        pltpu.make_async_copy(s1_hbm.at[src_c.at[b]], s1c.at[b], sems[b]).wait()
        pltpu.make_async_copy(s2_hbm.at[dst_c.at[b]], s2c.at[b], sems[b]).wait()

    def process(b):
        rows_v = rows[b]

        def group(g, _):
            sl = pl.ds(g * 16, 16)
            x = s1c[b, sl] + s2c[b, sl]
            h = jnp.exp(jnp.maximum(x, _SLOPE * x))
            h_v[sl] = h
            for e in range(16):
                he = h.at[jnp.full((16,), e, jnp.int32)].get(
                    mode="promise_in_bounds")
                for j in range(_D // 16):
                    rsl = pl.ds(j * 16, 16)
                    rows_v[g * 16 + e, rsl] = rows_v[g * 16 + e, rsl] * he
            return 0

        lax.fori_loop(0, _GRP, group, 0)
        pltpu.sync_copy(rows_v, acc_emb.at[src_c.at[b]], add=True)
        pltpu.sync_copy(h_v, acc_h.at[src_c.at[b]], add=True)

    unpack(0, 0)
    start_gathers(0)

    def pair(p, _):
        for t in range(2):
            k = 2 * p + t
            unpack(k + 1, 1 - t)
            start_gathers(1 - t)
            wait_gathers(t)
            process(t)
        return 0

    lax.fori_loop(0, (_NCHUNK - 1) // 2, pair, 0)
    wait_gathers(0)
    process(0)
    plsc.subcore_barrier()

    def copy_out(i, _):
        off = s * _ZROWS + i * _C
        pltpu.sync_copy(acc_emb.at[pl.ds(off, _C), :],
                        out_emb.at[c, pl.ds(off, _C), :])
        pltpu.sync_copy(acc_h.at[pl.ds(off, _C)], h_v)
        pltpu.sync_copy(h_v, out_h.at[pl.ds(c * _N + off, _C)])
        return 0

    lax.fori_loop(0, ncopies, copy_out, 0)


_sc_agg = functools.partial(
    pl.kernel,
    out_type=[
        jax.ShapeDtypeStruct((_NC, _N, _D), jnp.float32),
        jax.ShapeDtypeStruct((_NC * _N,), jnp.float32),
    ],
    mesh=plsc.VectorSubcoreMesh(core_axis_name="c", subcore_axis_name="s"),
    compiler_params=pltpu.CompilerParams(needs_layout_passes=False),
    scratch_types=[
        pltpu.VMEM((_NCHUNK, _C), jnp.int32),
        pltpu.VMEM((2, _C), jnp.int32),
        pltpu.VMEM((2, _C), jnp.int32),
        pltpu.VMEM((2, _C), jnp.float32),
        pltpu.VMEM((2, _C), jnp.float32),
        pltpu.VMEM((_C, _D), jnp.float32),
        pltpu.VMEM((_C, _D), jnp.float32),
        pltpu.VMEM((_C,), jnp.float32),
        pltpu.VMEM_SHARED((_N, _D), jnp.float32),
        pltpu.VMEM_SHARED((_N,), jnp.float32),
        pltpu.SemaphoreType.DMA,
        pltpu.SemaphoreType.DMA,
    ],
)(_sc_agg_body)


def kernel(nodes, edge_index, local_features, W, b, a):
    x = local_features.astype(jnp.float32)
    W = W.astype(jnp.float32)
    b2 = b.astype(jnp.float32).reshape(1, _D)
    a1 = a.astype(jnp.float32)[:_D].reshape(_D, 1)
    a2 = a.astype(jnp.float32)[_D:].reshape(_D, 1)
    src = edge_index[0].astype(jnp.int32)
    dst = edge_index[1].astype(jnp.int32)
    sd = (jnp.left_shift(src, 16) | dst).reshape(_NW, _NCHUNK, _C)

    emb, s1, s2 = pl.pallas_call(
        _embed_body,
        out_shape=[
            jax.ShapeDtypeStruct((_N, _D), jnp.float32),
            jax.ShapeDtypeStruct((_N, 1), jnp.float32),
            jax.ShapeDtypeStruct((_N, 1), jnp.float32),
        ],
    )(x, W, b2, a1, a2)

    s1f = s1.reshape(_N)
    s2f = s2.reshape(_N)
    part_emb, part_h = _sc_agg(emb, s1f, s2f, sd)

    out = pl.pallas_call(
        _combine_body,
        out_shape=jax.ShapeDtypeStruct((_N, _D), jnp.float32),
    )(part_emb, part_h.reshape(_NC, _N, 1), emb, s1, s2)
    return out


# speedup vs baseline: 16.1836x; 1.0150x over previous
import functools

import jax
import jax.numpy as jnp
from jax import lax
from jax.experimental import pallas as pl
from jax.experimental.pallas import tpu as pltpu
from jax.experimental.pallas import tpu_sc as plsc

_N = 10000
_E = 320000
_D = 128
_SLOPE = 0.1

_NC = 2
_NS = 16
_NW = _NC * _NS
_EPW = _E // _NW
_C = 80
_NCHUNK = _EPW // _C
_GRP = _C // 16
_ZROWS = 640


def _embed_body(x_ref, w_ref, b_ref, a1_ref, a2_ref, emb_ref, s1_ref, s2_ref):
    emb = jnp.dot(x_ref[...], w_ref[...], preferred_element_type=jnp.float32)
    emb = emb + b_ref[...]
    emb_ref[...] = emb
    s1_ref[...] = jnp.dot(emb, a1_ref[...], preferred_element_type=jnp.float32)
    s2_ref[...] = jnp.dot(emb, a2_ref[...], preferred_element_type=jnp.float32)


def _combine_body(pe_ref, ph_ref, emb_ref, s1_ref, s2_ref, out_ref):
    x = s1_ref[...] + s2_ref[...]
    hs = jnp.exp(jnp.maximum(x, _SLOPE * x))
    num = pe_ref[0, :, :] + pe_ref[1, :, :] + hs * emb_ref[...]
    den = ph_ref[0, :, :] + ph_ref[1, :, :] + hs
    out_ref[...] = num / den


def _sc_agg_body(emb_hbm, s1_hbm, s2_hbm, sd_hbm,
                 out_emb, out_h,
                 sd_a, src_c, dst_c, s1c, s2c, rows0, rows1, h_v,
                 acc_emb, acc_h, sem0, sem1, ssem0, ssem1):
    c = lax.axis_index("c")
    s = lax.axis_index("s")
    wid = s * _NC + c
    rows = (rows0, rows1)
    sems = (sem0, sem1)
    ssems = (ssem0, ssem1)

    zero16 = jnp.zeros((16,), jnp.float32)

    def zrow(i, _):
        for j in range(_D // 16):
            rows0[i, pl.ds(j * 16, 16)] = zero16
        return 0

    lax.fori_loop(0, _C, zrow, 0)
    for g in range(_GRP):
        h_v[0, pl.ds(g * 16, 16)] = zero16

    ncopies = jnp.minimum(_ZROWS, jnp.maximum(0, _N - s * _ZROWS)) // _C

    def zacc(i, _):
        off = s * _ZROWS + i * _C
        pltpu.sync_copy(rows0, acc_emb.at[pl.ds(off, _C), :])
        pltpu.sync_copy(h_v.at[0], acc_h.at[pl.ds(off, _C)])
        return 0

    lax.fori_loop(0, ncopies, zacc, 0)

    pltpu.sync_copy(sd_hbm.at[wid], sd_a)
    plsc.subcore_barrier()

    def unpack(k, b):
        for g in range(_GRP):
            sl = pl.ds(g * 16, 16)
            w = sd_a[k, sl]
            src_c[b, sl] = lax.shift_right_logical(w, 16)
            dst_c[b, sl] = jnp.bitwise_and(w, 0xFFFF)

    def start_gathers(b):
        pltpu.async_copy(emb_hbm.at[dst_c.at[b]], rows[b], sems[b])
        pltpu.async_copy(s1_hbm.at[src_c.at[b]], s1c.at[b], sems[b])
        pltpu.async_copy(s2_hbm.at[dst_c.at[b]], s2c.at[b], sems[b])

    def wait_gathers(b):
        pltpu.make_async_copy(emb_hbm.at[dst_c.at[b]], rows[b], sems[b]).wait()
        pltpu.make_async_copy(s1_hbm.at[src_c.at[b]], s1c.at[b], sems[b]).wait()
        pltpu.make_async_copy(s2_hbm.at[dst_c.at[b]], s2c.at[b], sems[b]).wait()

    def compute(b):
        rows_v = rows[b]

        def group(g, _):
            sl = pl.ds(g * 16, 16)
            x = s1c[b, sl] + s2c[b, sl]
            h = jnp.exp(jnp.maximum(x, _SLOPE * x))
            h_v[b, sl] = h
            for e in range(16):
                he = h.at[jnp.full((16,), e, jnp.int32)].get(
                    mode="promise_in_bounds")
                for j in range(_D // 16):
                    rsl = pl.ds(j * 16, 16)
                    rows_v[g * 16 + e, rsl] = rows_v[g * 16 + e, rsl] * he
            return 0

        lax.fori_loop(0, _GRP, group, 0)

    def start_scatter(b):
        pltpu.async_copy(rows[b], acc_emb.at[src_c.at[b]], ssems[b], add=True)
        pltpu.async_copy(h_v.at[b], acc_h.at[src_c.at[b]], ssems[b], add=True)

    def wait_scatter(b):
        pltpu.make_async_copy(rows[b], acc_emb.at[src_c.at[b]], ssems[b]).wait()
        pltpu.make_async_copy(h_v.at[b], acc_h.at[src_c.at[b]], ssems[b]).wait()

    unpack(0, 0)
    start_gathers(0)
    unpack(1, 1)
    start_gathers(1)
    wait_gathers(0)
    compute(0)
    start_scatter(0)

    def pair(p, _):
        for b in (1, 0):
            k = 2 * p + 1 + (1 - b)
            nb = 1 - b
            wait_scatter(nb)
            unpack(k + 1, nb)
            start_gathers(nb)
            wait_gathers(b)
            compute(b)
            start_scatter(b)
        return 0

    lax.fori_loop(0, (_NCHUNK - 3) // 2, pair, 0)
    wait_scatter(0)
    unpack(_NCHUNK - 1, 0)
    start_gathers(0)
    wait_gathers(1)
    compute(1)
    start_scatter(1)
    wait_scatter(1)
    wait_gathers(0)
    compute(0)
    start_scatter(0)
    wait_scatter(0)
    plsc.subcore_barrier()

    def copy_out(i, _):
        off = s * _ZROWS + i * _C
        pltpu.sync_copy(acc_emb.at[pl.ds(off, _C), :],
                        out_emb.at[c, pl.ds(off, _C), :])
        pltpu.sync_copy(acc_h.at[pl.ds(off, _C)], h_v.at[0])
        pltpu.sync_copy(h_v.at[0], out_h.at[pl.ds(c * _N + off, _C)])
        return 0

    lax.fori_loop(0, ncopies, copy_out, 0)


_sc_agg = functools.partial(
    pl.kernel,
    out_type=[
        jax.ShapeDtypeStruct((_NC, _N, _D), jnp.float32),
        jax.ShapeDtypeStruct((_NC * _N,), jnp.float32),
    ],
    mesh=plsc.VectorSubcoreMesh(core_axis_name="c", subcore_axis_name="s"),
    compiler_params=pltpu.CompilerParams(needs_layout_passes=False),
    scratch_types=[
        pltpu.VMEM((_NCHUNK, _C), jnp.int32),
        pltpu.VMEM((2, _C), jnp.int32),
        pltpu.VMEM((2, _C), jnp.int32),
        pltpu.VMEM((2, _C), jnp.float32),
        pltpu.VMEM((2, _C), jnp.float32),
        pltpu.VMEM((_C, _D), jnp.float32),
        pltpu.VMEM((_C, _D), jnp.float32),
        pltpu.VMEM((2, _C), jnp.float32),
        pltpu.VMEM_SHARED((_N, _D), jnp.float32),
        pltpu.VMEM_SHARED((_N,), jnp.float32),
        pltpu.SemaphoreType.DMA,
        pltpu.SemaphoreType.DMA,
        pltpu.SemaphoreType.DMA,
        pltpu.SemaphoreType.DMA,
    ],
)(_sc_agg_body)


def kernel(nodes, edge_index, local_features, W, b, a):
    x = local_features.astype(jnp.float32)
    W = W.astype(jnp.float32)
    b2 = b.astype(jnp.float32).reshape(1, _D)
    a1 = a.astype(jnp.float32)[:_D].reshape(_D, 1)
    a2 = a.astype(jnp.float32)[_D:].reshape(_D, 1)
    src = edge_index[0].astype(jnp.int32)
    dst = edge_index[1].astype(jnp.int32)
    sd = (jnp.left_shift(src, 16) | dst).reshape(_NW, _NCHUNK, _C)

    emb, s1, s2 = pl.pallas_call(
        _embed_body,
        out_shape=[
            jax.ShapeDtypeStruct((_N, _D), jnp.float32),
            jax.ShapeDtypeStruct((_N, 1), jnp.float32),
            jax.ShapeDtypeStruct((_N, 1), jnp.float32),
        ],
    )(x, W, b2, a1, a2)

    s1f = s1.reshape(_N)
    s2f = s2.reshape(_N)
    part_emb, part_h = _sc_agg(emb, s1f, s2f, sd)

    out = pl.pallas_call(
        _combine_body,
        out_shape=jax.ShapeDtypeStruct((_N, _D), jnp.float32),
    )(part_emb, part_h.reshape(_NC, _N, 1), emb, s1, s2)
    return out


# speedup vs baseline: 16.2813x; 1.0060x over previous
import functools

import jax
import jax.numpy as jnp
from jax import lax
from jax.experimental import pallas as pl
from jax.experimental.pallas import tpu as pltpu
from jax.experimental.pallas import tpu_sc as plsc

_N = 10000
_E = 320000
_D = 128
_SLOPE = 0.1

_NC = 2
_NS = 16
_NW = _NC * _NS
_EPW = _E // _NW
_C = 80
_NCHUNK = _EPW // _C
_GRP = _C // 16
_ZROWS = 640


def _embed_body(x_ref, w_ref, b_ref, a1_ref, a2_ref, emb_ref, s1_ref, s2_ref):
    emb = jnp.dot(x_ref[...], w_ref[...], preferred_element_type=jnp.float32)
    emb = emb + b_ref[...]
    emb_ref[...] = emb
    s1_ref[...] = jnp.dot(emb, a1_ref[...], preferred_element_type=jnp.float32)
    s2_ref[...] = jnp.dot(emb, a2_ref[...], preferred_element_type=jnp.float32)


def _combine_body(pe_ref, ph_ref, emb_ref, s1_ref, s2_ref, out_ref):
    x = s1_ref[...] + s2_ref[...]
    hs = jnp.exp(jnp.maximum(x, _SLOPE * x))
    num = pe_ref[0, :, :] + pe_ref[1, :, :] + hs * emb_ref[...]
    den = ph_ref[0, :, :] + ph_ref[1, :, :] + hs
    out_ref[...] = num / den


def _sc_agg_body(emb_hbm, s1_hbm, s2_hbm, sd_hbm,
                 out_emb, out_h,
                 sd_a, src_c, dst_c, s1c, s2c, rows0, rows1, h_v,
                 acc_emb, acc_h, sem0, sem1, ssem0, ssem1):
    c = lax.axis_index("c")
    s = lax.axis_index("s")
    wid = s * _NC + c
    rows = (rows0, rows1)
    sems = (sem0, sem1)
    ssems = (ssem0, ssem1)

    zero16 = jnp.zeros((16,), jnp.float32)

    def zrow(i, _):
        for j in range(_D // 16):
            rows0[i, pl.ds(j * 16, 16)] = zero16
        return 0

    lax.fori_loop(0, _C, zrow, 0)
    for g in range(_GRP):
        h_v[0, pl.ds(g * 16, 16)] = zero16

    ncopies = jnp.minimum(_ZROWS, jnp.maximum(0, _N - s * _ZROWS)) // _C

    def zacc(i, _):
        off = s * _ZROWS + i * _C
        pltpu.sync_copy(rows0, acc_emb.at[pl.ds(off, _C), :])
        pltpu.sync_copy(h_v.at[0], acc_h.at[pl.ds(off, _C)])
        return 0

    lax.fori_loop(0, ncopies, zacc, 0)

    pltpu.sync_copy(sd_hbm.at[wid], sd_a)
    plsc.subcore_barrier()

    def unpack(k, b):
        for g in range(_GRP):
            sl = pl.ds(g * 16, 16)
            w = sd_a[k, sl]
            src_c[b, sl] = lax.shift_right_logical(w, 16)
            dst_c[b, sl] = jnp.bitwise_and(w, 0xFFFF)

    def start_gathers(b):
        pltpu.async_copy(emb_hbm.at[dst_c.at[b]], rows[b], sems[b])
        pltpu.async_copy(s1_hbm.at[src_c.at[b]], s1c.at[b], sems[b])
        pltpu.async_copy(s2_hbm.at[dst_c.at[b]], s2c.at[b], sems[b])

    def wait_gathers(b):
        pltpu.make_async_copy(emb_hbm.at[dst_c.at[b]], rows[b], sems[b]).wait()
        pltpu.make_async_copy(s1_hbm.at[src_c.at[b]], s1c.at[b], sems[b]).wait()
        pltpu.make_async_copy(s2_hbm.at[dst_c.at[b]], s2c.at[b], sems[b]).wait()

    def compute(b):
        rows_v = rows[b]

        @plsc.parallel_loop(0, _GRP, 1, unroll=1)
        def group(g):
            sl = pl.ds(g * 16, 16)
            x = s1c[b, sl] + s2c[b, sl]
            h = jnp.exp(jnp.maximum(x, _SLOPE * x))
            h_v[b, sl] = h
            for e in range(16):
                he = h.at[jnp.full((16,), e, jnp.int32)].get(
                    mode="promise_in_bounds")
                for j in range(_D // 16):
                    rsl = pl.ds(j * 16, 16)
                    rows_v[g * 16 + e, rsl] = rows_v[g * 16 + e, rsl] * he

    def start_scatter(b):
        pltpu.async_copy(rows[b], acc_emb.at[src_c.at[b]], ssems[b], add=True)
        pltpu.async_copy(h_v.at[b], acc_h.at[src_c.at[b]], ssems[b], add=True)

    def wait_scatter(b):
        pltpu.make_async_copy(rows[b], acc_emb.at[src_c.at[b]], ssems[b]).wait()
        pltpu.make_async_copy(h_v.at[b], acc_h.at[src_c.at[b]], ssems[b]).wait()

    unpack(0, 0)
    start_gathers(0)
    unpack(1, 1)
    start_gathers(1)
    wait_gathers(0)
    compute(0)
    start_scatter(0)

    def pair(p, _):
        for b in (1, 0):
            k = 2 * p + 1 + (1 - b)
            nb = 1 - b
            wait_scatter(nb)
            unpack(k + 1, nb)
            start_gathers(nb)
            wait_gathers(b)
            compute(b)
            start_scatter(b)
        return 0

    lax.fori_loop(0, (_NCHUNK - 3) // 2, pair, 0)
    wait_scatter(0)
    unpack(_NCHUNK - 1, 0)
    start_gathers(0)
    wait_gathers(1)
    compute(1)
    start_scatter(1)
    wait_scatter(1)
    wait_gathers(0)
    compute(0)
    start_scatter(0)
    wait_scatter(0)
    plsc.subcore_barrier()

    def copy_out(i, _):
        off = s * _ZROWS + i * _C
        pltpu.sync_copy(acc_emb.at[pl.ds(off, _C), :],
                        out_emb.at[c, pl.ds(off, _C), :])
        pltpu.sync_copy(acc_h.at[pl.ds(off, _C)], h_v.at[0])
        pltpu.sync_copy(h_v.at[0], out_h.at[pl.ds(c * _N + off, _C)])
        return 0

    lax.fori_loop(0, ncopies, copy_out, 0)


_sc_agg = functools.partial(
    pl.kernel,
    out_type=[
        jax.ShapeDtypeStruct((_NC, _N, _D), jnp.float32),
        jax.ShapeDtypeStruct((_NC * _N,), jnp.float32),
    ],
    mesh=plsc.VectorSubcoreMesh(core_axis_name="c", subcore_axis_name="s"),
    compiler_params=pltpu.CompilerParams(needs_layout_passes=False),
    scratch_types=[
        pltpu.VMEM((_NCHUNK, _C), jnp.int32),
        pltpu.VMEM((2, _C), jnp.int32),
        pltpu.VMEM((2, _C), jnp.int32),
        pltpu.VMEM((2, _C), jnp.float32),
        pltpu.VMEM((2, _C), jnp.float32),
        pltpu.VMEM((_C, _D), jnp.float32),
        pltpu.VMEM((_C, _D), jnp.float32),
        pltpu.VMEM((2, _C), jnp.float32),
        pltpu.VMEM_SHARED((_N, _D), jnp.float32),
        pltpu.VMEM_SHARED((_N,), jnp.float32),
        pltpu.SemaphoreType.DMA,
        pltpu.SemaphoreType.DMA,
        pltpu.SemaphoreType.DMA,
        pltpu.SemaphoreType.DMA,
    ],
)(_sc_agg_body)


def kernel(nodes, edge_index, local_features, W, b, a):
    x = local_features.astype(jnp.float32)
    W = W.astype(jnp.float32)
    b2 = b.astype(jnp.float32).reshape(1, _D)
    a1 = a.astype(jnp.float32)[:_D].reshape(_D, 1)
    a2 = a.astype(jnp.float32)[_D:].reshape(_D, 1)
    src = edge_index[0].astype(jnp.int32)
    dst = edge_index[1].astype(jnp.int32)
    sd = (jnp.left_shift(src, 16) | dst).reshape(_NW, _NCHUNK, _C)

    emb, s1, s2 = pl.pallas_call(
        _embed_body,
        out_shape=[
            jax.ShapeDtypeStruct((_N, _D), jnp.float32),
            jax.ShapeDtypeStruct((_N, 1), jnp.float32),
            jax.ShapeDtypeStruct((_N, 1), jnp.float32),
        ],
    )(x, W, b2, a1, a2)

    s1f = s1.reshape(_N)
    s2f = s2.reshape(_N)
    part_emb, part_h = _sc_agg(emb, s1f, s2f, sd)

    out = pl.pallas_call(
        _combine_body,
        out_shape=jax.ShapeDtypeStruct((_N, _D), jnp.float32),
    )(part_emb, part_h.reshape(_NC, _N, 1), emb, s1, s2)
    return out
